# baseline (speedup 1.0000x reference)
import functools

import jax
import jax.numpy as jnp
from jax import lax
from jax.experimental import pallas as pl
from jax.experimental.pallas import tpu as pltpu
from jax.experimental.pallas import tpu_sc as plsc

NC = 2
NS = 16
NW = NC * NS

BATCH = 4096
HIST = 50
EMBED_DIM = 64

TOTAL = BATCH * HIST
PER_W = TOTAL // NW
CHUNK = 128
K = PER_W // CHUNK


def _gather_body(ids_hbm, table_hbm, out_hbm, idx_v, rows_v, gsem):
    wid = lax.axis_index("s") * NC + lax.axis_index("c")
    base = wid * PER_W
    pltpu.sync_copy(ids_hbm.at[wid], idx_v)

    def body(j, _):
        pltpu.async_copy(table_hbm.at[idx_v.at[j]], rows_v, gsem).wait()
        pltpu.sync_copy(rows_v, out_hbm.at[pl.ds(base + j * CHUNK, CHUNK)])
        return _

    lax.fori_loop(0, K, body, None)


@jax.jit
def _embed(ids3, table):
    mesh = plsc.VectorSubcoreMesh(core_axis_name="c", subcore_axis_name="s")
    run = pl.kernel(
        _gather_body,
        out_type=jax.ShapeDtypeStruct((TOTAL, EMBED_DIM), jnp.float32),
        mesh=mesh,
        scratch_types=[
            pltpu.VMEM((K, CHUNK), jnp.int32),
            pltpu.VMEM((CHUNK, EMBED_DIM), jnp.float32),
            pltpu.SemaphoreType.DMA,
        ],
        compiler_params=pltpu.CompilerParams(use_tc_tiling_on_sc=False),
    )
    return run(ids3, table)


def kernel(input_ids, embed_tokens_weight):
    ids3 = input_ids.astype(jnp.int32).reshape(NW, K, CHUNK)
    out = _embed(ids3, embed_tokens_weight)
    return out.reshape(BATCH, HIST, EMBED_DIM)


# speedup vs baseline: 1.0432x; 1.0432x over previous
import jax
import jax.numpy as jnp
from jax import lax
from jax.experimental import pallas as pl
from jax.experimental.pallas import tpu as pltpu
from jax.experimental.pallas import tpu_sc as plsc

NC = 2
NS = 16
NW = NC * NS

BATCH = 4096
HIST = 50
EMBED_DIM = 64

TOTAL = BATCH * HIST
PER_W = TOTAL // NW
CHUNK = 128
K = PER_W // CHUNK
PH = 5
ROWS = PH * CHUNK
NPH = K // PH


def _gather_body(ids_hbm, table_hbm, out_hbm, idx_v, big0, big1, gs0, gs1,
                 os0, os1):
    big = (big0, big1)
    gsem = (gs0, gs1)
    osem = (os0, os1)
    wid = lax.axis_index("s") * NC + lax.axis_index("c")
    base = wid * PER_W
    pltpu.sync_copy(ids_hbm.at[wid], idx_v)

    def fire(t, p):
        for c in range(PH):
            pltpu.async_copy(table_hbm.at[idx_v.at[t * PH + c]],
                             big[p].at[pl.ds(c * CHUNK, CHUNK)], gsem[p])

    def drain(t, p):
        for c in range(PH):
            pltpu.make_async_copy(table_hbm.at[idx_v.at[t * PH + c]],
                                  big[p].at[pl.ds(c * CHUNK, CHUNK)],
                                  gsem[p]).wait()

    def wstart(t, p):
        pltpu.async_copy(big[p], out_hbm.at[pl.ds(base + t * ROWS, ROWS)],
                         osem[p])

    def wwait(t, p):
        pltpu.make_async_copy(big[p],
                              out_hbm.at[pl.ds(base + t * ROWS, ROWS)],
                              osem[p]).wait()

    fire(0, 0)
    drain(0, 0)
    wstart(0, 0)
    fire(1, 1)
    drain(1, 1)
    wstart(1, 1)
    wwait(0, 0)
    fire(2, 0)

    def grp(q, _):
        t = 2 * q
        drain(t, 0)
        wstart(t, 0)
        wwait(t - 1, 1)
        fire(t + 1, 1)
        drain(t + 1, 1)
        wstart(t + 1, 1)
        wwait(t, 0)

        @pl.when(q < NPH // 2 - 1)
        def _():
            fire(t + 2, 0)

        return _

    lax.fori_loop(1, NPH // 2, grp, None)
    wwait(NPH - 1, 1)


@jax.jit
def _embed(ids3, table):
    mesh = plsc.VectorSubcoreMesh(core_axis_name="c", subcore_axis_name="s")
    run = pl.kernel(
        _gather_body,
        out_type=jax.ShapeDtypeStruct((TOTAL, EMBED_DIM), jnp.float32),
        mesh=mesh,
        scratch_types=[
            pltpu.VMEM((K, CHUNK), jnp.int32),
            pltpu.VMEM((ROWS, EMBED_DIM), jnp.float32),
            pltpu.VMEM((ROWS, EMBED_DIM), jnp.float32),
            pltpu.SemaphoreType.DMA,
            pltpu.SemaphoreType.DMA,
            pltpu.SemaphoreType.DMA,
            pltpu.SemaphoreType.DMA,
        ],
        compiler_params=pltpu.CompilerParams(use_tc_tiling_on_sc=False),
    )
    return run(ids3, table)


def kernel(input_ids, embed_tokens_weight):
    ids3 = input_ids.astype(jnp.int32).reshape(NW, K, CHUNK)
    out = _embed(ids3, embed_tokens_weight)
    return out.reshape(BATCH, HIST, EMBED_DIM)
